# baseline (speedup 1.0000x reference)
import functools

import jax
import jax.numpy as jnp
from jax import lax
from jax.experimental import pallas as pl
from jax.experimental.pallas import tpu as pltpu
from jax.experimental.pallas import tpu_sc as plsc

_B, _H, _W, _D = 8, 128, 128, 64
_ROW = _D * _W
_BROW = _H * _ROW
_NW = 32
_HPW = _H // 4
_RG = 4
_NG = _HPW // _RG


def _sc_sums_body(yt_hbm, yp_hbm, out_hbm, ytrow_v, ring_v, res_v, sems):
    wid = lax.axis_index("s") * 2 + lax.axis_index("c")
    b = wid // 4
    h0 = (wid % 4) * _HPW

    pltpu.sync_copy(yt_hbm.at[pl.ds(b * _BROW, _ROW)], ytrow_v)

    rowbase = b * _BROW + h0 * _ROW
    copies = [None, None]
    copies[0] = pltpu.async_copy(
        yp_hbm.at[pl.ds(rowbase, _RG * _ROW)], ring_v.at[0], sems.at[0]
    )

    for g in range(_NG):
        par = g % 2
        if g + 1 < _NG:
            copies[1 - par] = pltpu.async_copy(
                yp_hbm.at[pl.ds(rowbase + (g + 1) * _RG * _ROW, _RG * _ROW)],
                ring_v.at[1 - par],
                sems.at[1 - par],
            )
        copies[par].wait()

        zeros = jnp.zeros((16,), jnp.float32)
        init = (zeros,) * (2 * _RG)

        def jbody(j, carry, par=par):
            ytc = ytrow_v[pl.ds(j * 16, 16)]
            m1 = jnp.where(
                jnp.logical_and(ytc > 1000.0, ytc < 3000.0), 1.0, 0.0
            )
            m2 = jnp.where(
                jnp.logical_or(
                    jnp.logical_and(ytc > 0.0, ytc < 1000.0), ytc > 3000.0
                ),
                1.0,
                0.0,
            )
            out = []
            for r in range(_RG):
                yc = ring_v[par, pl.ds(r * _ROW + j * 16, 16)]
                out.append(carry[2 * r] + yc * m1)
                out.append(carry[2 * r + 1] + yc * m2)
            return tuple(out)

        accs = lax.fori_loop(0, _ROW // 16, jbody, init)
        for r in range(_RG):
            res_v[g * _RG + r, :] = accs[2 * r]
            res_v[_HPW + g * _RG + r, :] = accs[2 * r + 1]

    pltpu.sync_copy(res_v.at[pl.ds(0, _HPW)], out_hbm.at[b, pl.ds(h0, _HPW)])
    pltpu.sync_copy(
        res_v.at[pl.ds(_HPW, _HPW)], out_hbm.at[8 + b, pl.ds(h0, _HPW)]
    )


_sc_sums = functools.partial(
    pl.kernel,
    out_type=jax.ShapeDtypeStruct((16, _H, 16), jnp.float32),
    mesh=plsc.VectorSubcoreMesh(core_axis_name="c", subcore_axis_name="s"),
    scratch_types=[
        pltpu.VMEM((_ROW,), jnp.float32),
        pltpu.VMEM((2, _RG * _ROW), jnp.float32),
        pltpu.VMEM((2 * _HPW, 16), jnp.float32),
        pltpu.SemaphoreType.DMA((2,)),
    ],
)(_sc_sums_body)


def _corr_body(yt0_ref, part_ref, out_ref):
    s0 = yt0_ref[:, 0]
    m1 = jnp.logical_and(s0 > 1000.0, s0 < 3000.0).astype(jnp.float32)
    m2 = jnp.logical_or(
        jnp.logical_and(s0 > 0.0, s0 < 1000.0), s0 > 3000.0
    ).astype(jnp.float32)
    c1 = jnp.sum(m1)
    c2 = jnp.sum(m2)
    part = jnp.sum(part_ref[...], axis=2)
    a = jnp.sum(part[0:8], axis=0, keepdims=True) / c1
    bb = jnp.sum(part[8:16], axis=0, keepdims=True) / c2
    am = a - jnp.mean(a)
    bm = bb - jnp.mean(bb)
    cov = jnp.mean(am * bm)
    sx = jnp.sqrt(jnp.mean(am * am))
    sy = jnp.sqrt(jnp.mean(bm * bm))
    corr = cov / (sx * sy)
    out_ref[...] = jnp.abs(jnp.clip(corr, -1.0, 1.0)).reshape(1, 1)


def kernel(y_true, y_pred):
    yt = jnp.transpose(y_true[..., 0], (0, 1, 3, 2))
    yp = jnp.transpose(y_pred[..., 0], (0, 1, 3, 2))
    part = _sc_sums(yt.reshape(-1), yp.reshape(-1))
    out = pl.pallas_call(
        _corr_body,
        grid=(1,),
        in_specs=[
            pl.BlockSpec((_B, 1, _D, _W), lambda i: (0, 0, 0, 0)),
            pl.BlockSpec((16, _H, 16), lambda i: (0, 0, 0)),
        ],
        out_specs=pl.BlockSpec((1, 1), lambda i: (0, 0)),
        out_shape=jax.ShapeDtypeStruct((1, 1), jnp.float32),
    )(yt, part)
    return out


# speedup vs baseline: 1.5627x; 1.5627x over previous
import functools

import jax
import jax.numpy as jnp
from jax import lax
from jax.experimental import pallas as pl
from jax.experimental.pallas import tpu as pltpu
from jax.experimental.pallas import tpu_sc as plsc

_B, _H, _W, _D = 8, 128, 128, 64
_ROW = _D * _W
_BROW = _H * _ROW

_SCB = 2
_TCB = _B - _SCB
_NS = 2
_NB = _TCB // _NS

_WPB = 32 // _SCB
_HPW = _H // _WPB
_RG = 4
_NG = _HPW // _RG


def _sc_sums_body(yt_hbm, yp_hbm, out_hbm, ytrow_v, ring_v, res_v, sems):
    wid = lax.axis_index("s") * 2 + lax.axis_index("c")
    bi = wid // _WPB
    b = _TCB + bi
    h0 = (wid % _WPB) * _HPW

    pltpu.sync_copy(yt_hbm.at[pl.ds(b * _BROW, _ROW)], ytrow_v)

    rowbase = b * _BROW + h0 * _ROW
    copies = [None, None]
    copies[0] = pltpu.async_copy(
        yp_hbm.at[pl.ds(rowbase, _RG * _ROW)], ring_v.at[0], sems.at[0]
    )

    for g in range(_NG):
        par = g % 2
        if g + 1 < _NG:
            copies[1 - par] = pltpu.async_copy(
                yp_hbm.at[pl.ds(rowbase + (g + 1) * _RG * _ROW, _RG * _ROW)],
                ring_v.at[1 - par],
                sems.at[1 - par],
            )
        copies[par].wait()

        zeros = jnp.zeros((16,), jnp.float32)
        init = (zeros,) * (2 * _RG)

        def jbody(j, carry, par=par):
            ytc = ytrow_v[pl.ds(j * 16, 16)]
            m1 = jnp.where(
                jnp.logical_and(ytc > 1000.0, ytc < 3000.0), 1.0, 0.0
            )
            m2 = jnp.where(
                jnp.logical_or(
                    jnp.logical_and(ytc > 0.0, ytc < 1000.0), ytc > 3000.0
                ),
                1.0,
                0.0,
            )
            out = []
            for r in range(_RG):
                yc = ring_v[par, pl.ds(r * _ROW + j * 16, 16)]
                out.append(carry[2 * r] + yc * m1)
                out.append(carry[2 * r + 1] + yc * m2)
            return tuple(out)

        accs = lax.fori_loop(0, _ROW // 16, jbody, init)
        for r in range(_RG):
            res_v[g * _RG + r, :] = accs[2 * r]
            res_v[_HPW + g * _RG + r, :] = accs[2 * r + 1]

    pltpu.sync_copy(
        res_v.at[pl.ds(0, _HPW)], out_hbm.at[0, bi, pl.ds(h0, _HPW)]
    )
    pltpu.sync_copy(
        res_v.at[pl.ds(_HPW, _HPW)], out_hbm.at[1, bi, pl.ds(h0, _HPW)]
    )


_sc_sums = functools.partial(
    pl.kernel,
    out_type=jax.ShapeDtypeStruct((2, _SCB, _H, 16), jnp.float32),
    mesh=plsc.VectorSubcoreMesh(core_axis_name="c", subcore_axis_name="s"),
    scratch_types=[
        pltpu.VMEM((_ROW,), jnp.float32),
        pltpu.VMEM((2, _RG * _ROW), jnp.float32),
        pltpu.VMEM((2 * _HPW, 16), jnp.float32),
        pltpu.SemaphoreType.DMA((2,)),
    ],
)(_sc_sums_body)


def _masks_of(s0):
    m1 = jnp.logical_and(s0 > 1000.0, s0 < 3000.0).astype(jnp.float32)
    m2 = jnp.logical_or(
        jnp.logical_and(s0 > 0.0, s0 < 1000.0), s0 > 3000.0
    ).astype(jnp.float32)
    return m1, m2


def _tc_sums_body(*refs):
    yt_refs = refs[:_NS]
    yp_refs = refs[_NS:2 * _NS]
    acc1_ref, acc2_ref = refs[2 * _NS:]
    b = pl.program_id(0)

    @pl.when(b == 0)
    def _zero():
        acc1_ref[...] = jnp.zeros((_H, _W), jnp.float32)
        acc2_ref[...] = jnp.zeros((_H, _W), jnp.float32)

    p1 = jnp.zeros((_H, _W), jnp.float32)
    p2 = jnp.zeros((_H, _W), jnp.float32)
    for yt_ref, yp_ref in zip(yt_refs, yp_refs):
        m1, m2 = _masks_of(yt_ref[0, 0])
        ypv = yp_ref[0]
        p1 = p1 + jnp.sum(ypv * m1[None], axis=1)
        p2 = p2 + jnp.sum(ypv * m2[None], axis=1)

    acc1_ref[...] += p1
    acc2_ref[...] += p2


def _corr_body(yt0_ref, acc1_ref, acc2_ref, scpart_ref, out_ref):
    s0 = yt0_ref[:, 0]
    m1 = jnp.logical_and(s0 > 1000.0, s0 < 3000.0).astype(jnp.float32)
    m2 = jnp.logical_or(
        jnp.logical_and(s0 > 0.0, s0 < 1000.0), s0 > 3000.0
    ).astype(jnp.float32)
    c1 = jnp.sum(m1)
    c2 = jnp.sum(m2)
    sc = scpart_ref[...]
    a = jnp.sum(acc1_ref[...], axis=1) + jnp.sum(sc[0], axis=(0, 2))
    bb = jnp.sum(acc2_ref[...], axis=1) + jnp.sum(sc[1], axis=(0, 2))
    a = a / c1
    bb = bb / c2
    am = a - jnp.mean(a)
    bm = bb - jnp.mean(bb)
    cov = jnp.mean(am * bm)
    sx = jnp.sqrt(jnp.mean(am * am))
    sy = jnp.sqrt(jnp.mean(bm * bm))
    corr = cov / (sx * sy)
    out_ref[...] = jnp.abs(jnp.clip(corr, -1.0, 1.0)).reshape(1, 1)


def kernel(y_true, y_pred):
    yt = jnp.transpose(y_true[..., 0], (0, 1, 3, 2))
    yp = jnp.transpose(y_pred[..., 0], (0, 1, 3, 2))
    scpart = _sc_sums(yt.reshape(-1), yp.reshape(-1))
    acc1, acc2 = pl.pallas_call(
        _tc_sums_body,
        grid=(_NB,),
        in_specs=(
            [pl.BlockSpec((1, 1, _D, _W),
                          (lambda s: lambda b: (b + s * _NB, 0, 0, 0))(s))
             for s in range(_NS)]
            + [pl.BlockSpec((1, _H, _D, _W),
                            (lambda s: lambda b: (b + s * _NB, 0, 0, 0))(s))
               for s in range(_NS)]
        ),
        out_specs=[
            pl.BlockSpec((_H, _W), lambda b: (0, 0)),
            pl.BlockSpec((_H, _W), lambda b: (0, 0)),
        ],
        out_shape=[
            jax.ShapeDtypeStruct((_H, _W), jnp.float32),
            jax.ShapeDtypeStruct((_H, _W), jnp.float32),
        ],
    )(*([yt] * _NS + [yp] * _NS))
    out = pl.pallas_call(
        _corr_body,
        grid=(1,),
        in_specs=[
            pl.BlockSpec((_B, 1, _D, _W), lambda i: (0, 0, 0, 0)),
            pl.BlockSpec((_H, _W), lambda i: (0, 0)),
            pl.BlockSpec((_H, _W), lambda i: (0, 0)),
            pl.BlockSpec((2, _SCB, _H, 16), lambda i: (0, 0, 0, 0)),
        ],
        out_specs=pl.BlockSpec((1, 1), lambda i: (0, 0)),
        out_shape=jax.ShapeDtypeStruct((1, 1), jnp.float32),
    )(yt, acc1, acc2, scpart)
    return out


# speedup vs baseline: 3.1619x; 2.0233x over previous
import jax
import jax.numpy as jnp
from jax.experimental import pallas as pl
from jax.experimental.pallas import tpu as pltpu

_B, _H, _W, _D = 8, 128, 128, 64
_NS = 2
_NB = _B // _NS


def _masks_of(s0):
    m1 = jnp.logical_and(s0 > 1000.0, s0 < 3000.0).astype(jnp.float32)
    m2 = jnp.logical_or(
        jnp.logical_and(s0 > 0.0, s0 < 1000.0), s0 > 3000.0
    ).astype(jnp.float32)
    return m1, m2


def _sums_body(*refs):
    yt_refs = refs[:_NS]
    yp_refs = refs[_NS:2 * _NS]
    acc1_ref, acc2_ref, cnt_ref = refs[2 * _NS:]
    b = pl.program_id(0)

    @pl.when(b == 0)
    def _zero():
        acc1_ref[...] = jnp.zeros((_H, _W), jnp.float32)
        acc2_ref[...] = jnp.zeros((_H, _W), jnp.float32)
        cnt_ref[...] = jnp.zeros((1, 128), jnp.float32)

    p1 = jnp.zeros((_H, _W), jnp.float32)
    p2 = jnp.zeros((_H, _W), jnp.float32)
    c1 = jnp.float32(0.0)
    c2 = jnp.float32(0.0)
    for yt_ref, yp_ref in zip(yt_refs, yp_refs):
        m1, m2 = _masks_of(yt_ref[0, 0])
        ypv = yp_ref[0]
        p1 = p1 + jnp.sum(ypv * m1[None], axis=1)
        p2 = p2 + jnp.sum(ypv * m2[None], axis=1)
        c1 = c1 + jnp.sum(m1)
        c2 = c2 + jnp.sum(m2)

    acc1_ref[...] += p1
    acc2_ref[...] += p2

    lane = jax.lax.broadcasted_iota(jnp.int32, (1, 128), 1)
    cnt_ref[...] += jnp.where(lane == 0, c1, 0.0) + jnp.where(lane == 1, c2, 0.0)


def _corr_body(acc1_ref, acc2_ref, cnt_ref, out_ref):
    a = jnp.sum(acc1_ref[...], axis=1, keepdims=True) / cnt_ref[0, 0]
    bb = jnp.sum(acc2_ref[...], axis=1, keepdims=True) / cnt_ref[0, 1]
    am = a - jnp.mean(a)
    bm = bb - jnp.mean(bb)
    cov = jnp.mean(am * bm)
    sx = jnp.sqrt(jnp.mean(am * am))
    sy = jnp.sqrt(jnp.mean(bm * bm))
    corr = cov / (sx * sy)
    out_ref[...] = jnp.abs(jnp.clip(corr, -1.0, 1.0)).reshape(1, 1)


def kernel(y_true, y_pred):
    yt = jnp.transpose(y_true[..., 0], (0, 1, 3, 2))
    yp = jnp.transpose(y_pred[..., 0], (0, 1, 3, 2))
    acc1, acc2, cnt = pl.pallas_call(
        _sums_body,
        grid=(_NB,),
        in_specs=(
            [pl.BlockSpec((1, 1, _D, _W),
                          (lambda s: lambda b: (b + s * _NB, 0, 0, 0))(s))
             for s in range(_NS)]
            + [pl.BlockSpec((1, _H, _D, _W),
                            (lambda s: lambda b: (b + s * _NB, 0, 0, 0))(s))
               for s in range(_NS)]
        ),
        out_specs=[
            pl.BlockSpec((_H, _W), lambda b: (0, 0)),
            pl.BlockSpec((_H, _W), lambda b: (0, 0)),
            pl.BlockSpec((1, 128), lambda b: (0, 0)),
        ],
        out_shape=[
            jax.ShapeDtypeStruct((_H, _W), jnp.float32),
            jax.ShapeDtypeStruct((_H, _W), jnp.float32),
            jax.ShapeDtypeStruct((1, 128), jnp.float32),
        ],
    )(*([yt] * _NS + [yp] * _NS))
    out = pl.pallas_call(
        _corr_body,
        out_shape=jax.ShapeDtypeStruct((1, 1), jnp.float32),
    )(acc1, acc2, cnt)
    return out
